# baseline (speedup 1.0000x reference)
import functools

import jax
import jax.numpy as jnp
from jax.experimental import pallas as pl
from jax.experimental.pallas import tpu as pltpu
from jax.experimental.pallas import tpu_sc as plsc

N_TOKEN = 100000
D_PROJ = 1024
CUT0, CUT1, CUT2 = 20000, 60000, 100000
D0, D1, D2 = 1024, 256, 64
N_TOK_TOTAL = 8192

GATHER_WINDOW = 128
TOK_BLOCK = 512

SPLIT0 = 4
SPLIT1 = 2


def _sc_gather(w0, w1, w2, i0q, i1d, i2):
    mesh = plsc.VectorSubcoreMesh(core_axis_name="c", subcore_axis_name="s")

    out_type = (
        jax.ShapeDtypeStruct((N_TOK_TOTAL * SPLIT0, D0 // SPLIT0), jnp.float32),
        jax.ShapeDtypeStruct((N_TOK_TOTAL * SPLIT1, D1 // SPLIT1), jnp.float32),
        jax.ShapeDtypeStruct((N_TOK_TOTAL, 2 * D2), jnp.float32),
    )

    @functools.partial(pl.kernel, out_type=out_type, mesh=mesh)
    def gather_kernel(w0h, w1h, w2h, i0h, i1h, i2h, e0h, e1h, e2h):
        def body0(iv, ov):
            pltpu.sync_copy(w0h.at[iv.at[0]], ov)

        def body1(iv, ov):
            pltpu.sync_copy(w1h.at[iv.at[0]], ov)

        def body2(iv, ov):
            pltpu.sync_copy(w2h.at[iv.at[0]], ov)

        for body, ih, eh, n_idx, d in (
            (body0, i0h, e0h, N_TOK_TOTAL * SPLIT0, D0 // SPLIT0),
            (body1, i1h, e1h, N_TOK_TOTAL * SPLIT1, D1 // SPLIT1),
            (body2, i2h, e2h, N_TOK_TOTAL, 2 * D2),
        ):
            pltpu.emit_pipeline(
                body,
                grid=(n_idx // GATHER_WINDOW,),
                in_specs=[pl.BlockSpec((1, GATHER_WINDOW), lambda i: (0, i))],
                out_specs=[pl.BlockSpec((GATHER_WINDOW, d), lambda i: (i, 0))],
                core_axis_name=("c", "s"),
                dimension_semantics=(pltpu.PARALLEL,),
            )(ih, eh)

    return gather_kernel(w0.reshape(-1, D0 // SPLIT0),
                         w1.reshape(-1, D1 // SPLIT1),
                         w2.reshape(-1, 2 * D2), i0q, i1d, i2)


def _tc_project_body(inp_ref, e0_ref, e1_ref, e2_ref, p0_ref, p1_ref, p2_ref,
                     out_ref):
    tok = inp_ref[...]
    m0 = tok < CUT0
    m1 = jnp.logical_and(tok >= CUT0, tok < CUT1)
    m2 = tok >= CUT1
    odd = jnp.equal(jnp.bitwise_and(tok, 1), 1)
    e2pair = e2_ref[...]
    e2 = jnp.where(odd, e2pair[:, D2:], e2pair[:, :D2])
    a0 = jnp.where(m0, e0_ref[...], 0.0).astype(jnp.bfloat16)
    a1 = jnp.where(m1, e1_ref[...], 0.0).astype(jnp.bfloat16)
    a2 = jnp.where(m2, e2, 0.0).astype(jnp.bfloat16)
    acc = jnp.dot(a0, p0_ref[...], preferred_element_type=jnp.float32)
    acc = acc + jnp.dot(a1, p1_ref[...], preferred_element_type=jnp.float32)
    acc = acc + jnp.dot(a2, p2_ref[...], preferred_element_type=jnp.float32)
    out_ref[...] = acc * (D_PROJ ** 0.5)


def _tc_project(inp2, e0, e1, e2, p0b, p1b, p2b):
    grid = (N_TOK_TOTAL // TOK_BLOCK,)
    return pl.pallas_call(
        _tc_project_body,
        grid=grid,
        in_specs=[
            pl.BlockSpec((TOK_BLOCK, 1), lambda i: (i, 0)),
            pl.BlockSpec((TOK_BLOCK, D0), lambda i: (i, 0)),
            pl.BlockSpec((TOK_BLOCK, D1), lambda i: (i, 0)),
            pl.BlockSpec((TOK_BLOCK, 2 * D2), lambda i: (i, 0)),
            pl.BlockSpec((D0, D_PROJ), lambda i: (0, 0)),
            pl.BlockSpec((D1, D_PROJ), lambda i: (0, 0)),
            pl.BlockSpec((D2, D_PROJ), lambda i: (0, 0)),
        ],
        out_specs=pl.BlockSpec((TOK_BLOCK, D_PROJ), lambda i: (i, 0)),
        out_shape=jax.ShapeDtypeStruct((N_TOK_TOTAL, D_PROJ), jnp.float32),
        compiler_params=pltpu.CompilerParams(
            dimension_semantics=("parallel",)),
    )(inp2, e0, e1, e2, p0b, p1b, p2b)


def kernel(inp, w0, w1, w2, p0, p1, p2):
    inp_flat = inp.reshape(-1)
    i0 = jnp.clip(inp_flat, 0, CUT0 - 1)
    i1 = jnp.clip(inp_flat, CUT0, CUT1 - 1) - CUT0
    i2 = ((jnp.clip(inp_flat, CUT1, CUT2 - 1) - CUT1) >> 1).reshape(1, -1)
    i0q = (i0[:, None] * SPLIT0 + jnp.arange(SPLIT0, dtype=jnp.int32)
           ).reshape(1, -1)
    i1d = (i1[:, None] * SPLIT1 + jnp.arange(SPLIT1, dtype=jnp.int32)
           ).reshape(1, -1)

    e0, e1, e2 = _sc_gather(w0, w1, w2, i0q, i1d, i2)
    e0 = e0.reshape(N_TOK_TOTAL, D0)
    e1 = e1.reshape(N_TOK_TOTAL, D1)

    inp2 = inp_flat.reshape(-1, 1)
    out = _tc_project(inp2, e0, e1, e2,
                      p0.astype(jnp.bfloat16),
                      p1.astype(jnp.bfloat16),
                      p2.astype(jnp.bfloat16))
    return out.reshape(inp.shape + (D_PROJ,))


# speedup vs baseline: 1.0004x; 1.0004x over previous
import functools

import jax
import jax.numpy as jnp
from jax.experimental import pallas as pl
from jax.experimental.pallas import tpu as pltpu
from jax.experimental.pallas import tpu_sc as plsc

N_TOKEN = 100000
D_PROJ = 1024
CUT0, CUT1, CUT2 = 20000, 60000, 100000
D0, D1, D2 = 1024, 256, 64
N_TOK_TOTAL = 8192

TOK_BLOCK = 512

SPLIT0 = 4
SPLIT1 = 2

NC, NS = 2, 16
NW = NC * NS
TOK_W = N_TOK_TOTAL // NW

PLAN0 = (TOK_W * SPLIT0, 128, D0 // SPLIT0)
PLAN1 = (TOK_W * SPLIT1, 128, D1 // SPLIT1)
PLAN2 = (TOK_W, 64, 2 * D2)


def _sc_gather(w0, w1, w2, i0q, i1d, i2):
    mesh = plsc.VectorSubcoreMesh(core_axis_name="c", subcore_axis_name="s")

    out_type = (
        jax.ShapeDtypeStruct((N_TOK_TOTAL * SPLIT0, D0 // SPLIT0), jnp.float32),
        jax.ShapeDtypeStruct((N_TOK_TOTAL * SPLIT1, D1 // SPLIT1), jnp.float32),
        jax.ShapeDtypeStruct((N_TOK_TOTAL, 2 * D2), jnp.float32),
    )

    scratch_types = [
        pltpu.VMEM((PLAN0[0],), jnp.int32),
        pltpu.VMEM((PLAN1[0],), jnp.int32),
        pltpu.VMEM((PLAN2[0],), jnp.int32),
        pltpu.VMEM((PLAN0[1], PLAN0[2]), jnp.float32),
        pltpu.VMEM((PLAN0[1], PLAN0[2]), jnp.float32),
        pltpu.VMEM((PLAN1[1], PLAN1[2]), jnp.float32),
        pltpu.VMEM((PLAN1[1], PLAN1[2]), jnp.float32),
        pltpu.VMEM((PLAN2[1], PLAN2[2]), jnp.float32),
        pltpu.VMEM((PLAN2[1], PLAN2[2]), jnp.float32),
        pltpu.SemaphoreType.DMA,
        pltpu.SemaphoreType.DMA,
        pltpu.SemaphoreType.DMA,
        pltpu.SemaphoreType.DMA,
    ]

    @functools.partial(pl.kernel, out_type=out_type, mesh=mesh,
                       scratch_types=scratch_types)
    def gather_kernel(w0h, w1h, w2h, i0h, i1h, i2h, e0h, e1h, e2h,
                      i0v, i1v, i2v, b0a, b0b, b1a, b1b, b2a, b2b,
                      gsa, gsb, osa, osb):
        wid = jax.lax.axis_index("s") * NC + jax.lax.axis_index("c")

        def run_table(wh, ih, eh, iv, bufs, plan):
            n_sub, chunk, _ = plan
            n_chunks = n_sub // chunk
            base = wid * n_sub
            pltpu.sync_copy(ih.at[pl.ds(base, n_sub)], iv)
            gsems = (gsa, gsb)
            osems = (osa, osb)

            def fire_gather(c):
                b = c % 2
                return pltpu.async_copy(
                    wh.at[iv.at[pl.ds(c * chunk, chunk)]], bufs[b], gsems[b])

            def fire_out(c):
                b = c % 2
                return pltpu.async_copy(
                    bufs[b], eh.at[pl.ds(base + c * chunk, chunk)], osems[b])

            gh = [None] * n_chunks
            oh = [None] * n_chunks
            gh[0] = fire_gather(0)
            for c in range(n_chunks):
                if c + 1 < n_chunks:
                    if c >= 1:
                        oh[c - 1].wait()
                    gh[c + 1] = fire_gather(c + 1)
                gh[c].wait()
                oh[c] = fire_out(c)
            oh[n_chunks - 1].wait()
            if n_chunks >= 2:
                oh[n_chunks - 2].wait()

        run_table(w0h, i0h, e0h, i0v, (b0a, b0b), PLAN0)
        run_table(w1h, i1h, e1h, i1v, (b1a, b1b), PLAN1)
        run_table(w2h, i2h, e2h, i2v, (b2a, b2b), PLAN2)

    return gather_kernel(w0.reshape(-1, D0 // SPLIT0),
                         w1.reshape(-1, D1 // SPLIT1),
                         w2.reshape(-1, 2 * D2), i0q, i1d, i2)


def _tc_project_body(inp_ref, e0_ref, e1_ref, e2_ref, p0_ref, p1_ref, p2_ref,
                     out_ref):
    tok = inp_ref[...]
    m0 = tok < CUT0
    m1 = jnp.logical_and(tok >= CUT0, tok < CUT1)
    m2 = tok >= CUT1
    odd = jnp.equal(jnp.bitwise_and(tok, 1), 1)
    e2pair = e2_ref[...]
    e2 = jnp.where(odd, e2pair[:, D2:], e2pair[:, :D2])
    a0 = jnp.where(m0, e0_ref[...], 0.0).astype(jnp.bfloat16)
    a1 = jnp.where(m1, e1_ref[...], 0.0).astype(jnp.bfloat16)
    a2 = jnp.where(m2, e2, 0.0).astype(jnp.bfloat16)
    acc = jnp.dot(a0, p0_ref[...], preferred_element_type=jnp.float32)
    acc = acc + jnp.dot(a1, p1_ref[...], preferred_element_type=jnp.float32)
    acc = acc + jnp.dot(a2, p2_ref[...], preferred_element_type=jnp.float32)
    out_ref[...] = acc * (D_PROJ ** 0.5)


def _tc_project(inp2, e0, e1, e2, p0b, p1b, p2b):
    grid = (N_TOK_TOTAL // TOK_BLOCK,)
    return pl.pallas_call(
        _tc_project_body,
        grid=grid,
        in_specs=[
            pl.BlockSpec((TOK_BLOCK, 1), lambda i: (i, 0)),
            pl.BlockSpec((TOK_BLOCK, D0), lambda i: (i, 0)),
            pl.BlockSpec((TOK_BLOCK, D1), lambda i: (i, 0)),
            pl.BlockSpec((TOK_BLOCK, 2 * D2), lambda i: (i, 0)),
            pl.BlockSpec((D0, D_PROJ), lambda i: (0, 0)),
            pl.BlockSpec((D1, D_PROJ), lambda i: (0, 0)),
            pl.BlockSpec((D2, D_PROJ), lambda i: (0, 0)),
        ],
        out_specs=pl.BlockSpec((TOK_BLOCK, D_PROJ), lambda i: (i, 0)),
        out_shape=jax.ShapeDtypeStruct((N_TOK_TOTAL, D_PROJ), jnp.float32),
        compiler_params=pltpu.CompilerParams(
            dimension_semantics=("parallel",)),
    )(inp2, e0, e1, e2, p0b, p1b, p2b)


def kernel(inp, w0, w1, w2, p0, p1, p2):
    inp_flat = inp.reshape(-1)
    i0 = jnp.clip(inp_flat, 0, CUT0 - 1)
    i1 = jnp.clip(inp_flat, CUT0, CUT1 - 1) - CUT0
    i2 = ((jnp.clip(inp_flat, CUT1, CUT2 - 1) - CUT1) >> 1).reshape(-1)
    i0q = (i0[:, None] * SPLIT0 + jnp.arange(SPLIT0, dtype=jnp.int32)
           ).reshape(-1)
    i1d = (i1[:, None] * SPLIT1 + jnp.arange(SPLIT1, dtype=jnp.int32)
           ).reshape(-1)

    e0, e1, e2 = _sc_gather(w0, w1, w2, i0q, i1d, i2)
    e0 = e0.reshape(N_TOK_TOTAL, D0)
    e1 = e1.reshape(N_TOK_TOTAL, D1)

    inp2 = inp_flat.reshape(-1, 1)
    out = _tc_project(inp2, e0, e1, e2,
                      p0.astype(jnp.bfloat16),
                      p1.astype(jnp.bfloat16),
                      p2.astype(jnp.bfloat16))
    return out.reshape(inp.shape + (D_PROJ,))


# speedup vs baseline: 1.2640x; 1.2635x over previous
import functools

import jax
import jax.numpy as jnp
from jax.experimental import pallas as pl
from jax.experimental.pallas import tpu as pltpu
from jax.experimental.pallas import tpu_sc as plsc

N_TOKEN = 100000
D_PROJ = 1024
CUT0, CUT1, CUT2 = 20000, 60000, 100000
D0, D1, D2 = 1024, 256, 64
N_TOK_TOTAL = 8192

TOK_BLOCK = 512

NC, NS = 2, 16
NW = NC * NS
TOK_W = N_TOK_TOTAL // NW

PLAN0 = (TOK_W, 32, D0)
PLAN1 = (TOK_W, 64, D1)
PLAN2 = (TOK_W, 64, 2 * D2)


def _sc_gather(w0, w1, w2, i0q, i1d, i2):
    mesh = plsc.VectorSubcoreMesh(core_axis_name="c", subcore_axis_name="s")

    out_type = (
        jax.ShapeDtypeStruct((N_TOK_TOTAL, D0), jnp.float32),
        jax.ShapeDtypeStruct((N_TOK_TOTAL, D1), jnp.float32),
        jax.ShapeDtypeStruct((N_TOK_TOTAL, 2 * D2), jnp.float32),
    )

    scratch_types = [
        pltpu.VMEM((PLAN0[0],), jnp.int32),
        pltpu.VMEM((PLAN1[0],), jnp.int32),
        pltpu.VMEM((PLAN2[0],), jnp.int32),
        pltpu.VMEM((PLAN0[1], PLAN0[2]), jnp.float32),
        pltpu.VMEM((PLAN0[1], PLAN0[2]), jnp.float32),
        pltpu.VMEM((PLAN1[1], PLAN1[2]), jnp.float32),
        pltpu.VMEM((PLAN1[1], PLAN1[2]), jnp.float32),
        pltpu.VMEM((PLAN2[1], PLAN2[2]), jnp.float32),
        pltpu.VMEM((PLAN2[1], PLAN2[2]), jnp.float32),
        pltpu.SemaphoreType.DMA,
        pltpu.SemaphoreType.DMA,
        pltpu.SemaphoreType.DMA,
        pltpu.SemaphoreType.DMA,
    ]

    @functools.partial(pl.kernel, out_type=out_type, mesh=mesh,
                       scratch_types=scratch_types)
    def gather_kernel(w0h, w1h, w2h, i0h, i1h, i2h, e0h, e1h, e2h,
                      i0v, i1v, i2v, b0a, b0b, b1a, b1b, b2a, b2b,
                      gsa, gsb, osa, osb):
        wid = jax.lax.axis_index("s") * NC + jax.lax.axis_index("c")

        def run_table(wh, ih, eh, iv, bufs, plan):
            n_sub, chunk, _ = plan
            n_chunks = n_sub // chunk
            base = wid * n_sub
            pltpu.sync_copy(ih.at[pl.ds(base, n_sub)], iv)
            gsems = (gsa, gsb)
            osems = (osa, osb)

            def fire_gather(c):
                b = c % 2
                return pltpu.async_copy(
                    wh.at[iv.at[pl.ds(c * chunk, chunk)]], bufs[b], gsems[b])

            def fire_out(c):
                b = c % 2
                return pltpu.async_copy(
                    bufs[b], eh.at[pl.ds(base + c * chunk, chunk)], osems[b])

            gh = [None] * n_chunks
            oh = [None] * n_chunks
            gh[0] = fire_gather(0)
            for c in range(n_chunks):
                if c + 1 < n_chunks:
                    if c >= 1:
                        oh[c - 1].wait()
                    gh[c + 1] = fire_gather(c + 1)
                gh[c].wait()
                oh[c] = fire_out(c)
            oh[n_chunks - 1].wait()
            if n_chunks >= 2:
                oh[n_chunks - 2].wait()

        run_table(w0h, i0h, e0h, i0v, (b0a, b0b), PLAN0)
        run_table(w1h, i1h, e1h, i1v, (b1a, b1b), PLAN1)
        run_table(w2h, i2h, e2h, i2v, (b2a, b2b), PLAN2)

    return gather_kernel(w0, w1, w2.reshape(-1, 2 * D2), i0q, i1d, i2)


def _tc_project_body(inp_ref, e0_ref, e1_ref, e2_ref, p0_ref, p1_ref, p2_ref,
                     out_ref):
    tok = inp_ref[...]
    m0 = tok < CUT0
    m1 = jnp.logical_and(tok >= CUT0, tok < CUT1)
    m2 = tok >= CUT1
    odd = jnp.equal(jnp.bitwise_and(tok, 1), 1)
    e2pair = e2_ref[...]
    e2 = jnp.where(odd, e2pair[:, D2:], e2pair[:, :D2])
    a0 = jnp.where(m0, e0_ref[...], 0.0).astype(jnp.bfloat16)
    a1 = jnp.where(m1, e1_ref[...], 0.0).astype(jnp.bfloat16)
    a2 = jnp.where(m2, e2, 0.0).astype(jnp.bfloat16)
    acc = jnp.dot(a0, p0_ref[...], preferred_element_type=jnp.float32)
    acc = acc + jnp.dot(a1, p1_ref[...], preferred_element_type=jnp.float32)
    acc = acc + jnp.dot(a2, p2_ref[...], preferred_element_type=jnp.float32)
    out_ref[...] = acc * (D_PROJ ** 0.5)


def _tc_project(inp2, e0, e1, e2, p0b, p1b, p2b):
    grid = (N_TOK_TOTAL // TOK_BLOCK,)
    return pl.pallas_call(
        _tc_project_body,
        grid=grid,
        in_specs=[
            pl.BlockSpec((TOK_BLOCK, 1), lambda i: (i, 0)),
            pl.BlockSpec((TOK_BLOCK, D0), lambda i: (i, 0)),
            pl.BlockSpec((TOK_BLOCK, D1), lambda i: (i, 0)),
            pl.BlockSpec((TOK_BLOCK, 2 * D2), lambda i: (i, 0)),
            pl.BlockSpec((D0, D_PROJ), lambda i: (0, 0)),
            pl.BlockSpec((D1, D_PROJ), lambda i: (0, 0)),
            pl.BlockSpec((D2, D_PROJ), lambda i: (0, 0)),
        ],
        out_specs=pl.BlockSpec((TOK_BLOCK, D_PROJ), lambda i: (i, 0)),
        out_shape=jax.ShapeDtypeStruct((N_TOK_TOTAL, D_PROJ), jnp.float32),
        compiler_params=pltpu.CompilerParams(
            dimension_semantics=("parallel",)),
    )(inp2, e0, e1, e2, p0b, p1b, p2b)


def kernel(inp, w0, w1, w2, p0, p1, p2):
    inp_flat = inp.reshape(-1)
    i0 = jnp.clip(inp_flat, 0, CUT0 - 1)
    i1 = jnp.clip(inp_flat, CUT0, CUT1 - 1) - CUT0
    i2 = ((jnp.clip(inp_flat, CUT1, CUT2 - 1) - CUT1) >> 1).reshape(-1)

    e0, e1, e2 = _sc_gather(w0, w1, w2, i0, i1, i2)

    inp2 = inp_flat.reshape(-1, 1)
    out = _tc_project(inp2, e0, e1, e2,
                      p0.astype(jnp.bfloat16),
                      p1.astype(jnp.bfloat16),
                      p2.astype(jnp.bfloat16))
    return out.reshape(inp.shape + (D_PROJ,))


# speedup vs baseline: 7.9186x; 6.2646x over previous
import dataclasses
import functools

import jax
import jax.numpy as jnp
from jax.experimental import pallas as pl
from jax.experimental.pallas import tpu as pltpu
from jax.experimental.pallas import tpu_sc as plsc

N_TOKEN = 100000
D_PROJ = 1024
CUT0, CUT1, CUT2 = 20000, 60000, 100000
D0, D1, D2 = 1024, 256, 64
N_TOK_TOTAL = 8192

TOK_BLOCK = 512

NC, NS = 2, 16
NW = NC * NS
TOK_W = N_TOK_TOTAL // NW

CH = 64
NCHUNK = TOK_W // CH
CH_SHIFT, CH_MASK = 6, CH - 1


def _sc_gather(toks, w0, w1, w2):
    mesh = plsc.VectorSubcoreMesh(core_axis_name="c", subcore_axis_name="s")

    out_type = (
        jax.ShapeDtypeStruct((N_TOK_TOTAL, D0), jnp.float32),
        jax.ShapeDtypeStruct((N_TOK_TOTAL, D1), jnp.float32),
        jax.ShapeDtypeStruct((N_TOK_TOTAL, 2 * D2), jnp.float32),
    )

    scratch_types = [
        pltpu.VMEM((TOK_W,), jnp.int32),
        pltpu.VMEM((NCHUNK, CH), jnp.int32),
        pltpu.VMEM((NCHUNK, CH), jnp.int32),
        pltpu.VMEM((NCHUNK, CH), jnp.int32),
        pltpu.VMEM((NCHUNK, CH), jnp.int32),
        pltpu.VMEM((NCHUNK, CH), jnp.int32),
        pltpu.VMEM((NCHUNK, CH), jnp.int32),
        pltpu.VMEM((CH, D0), jnp.float32),
        pltpu.VMEM((CH, D1), jnp.float32),
        pltpu.VMEM((CH, 2 * D2), jnp.float32),
    ]

    cp = pltpu.CompilerParams()
    if "needs_layout_passes" in pltpu.CompilerParams.__dataclass_fields__:
        cp = dataclasses.replace(cp, needs_layout_passes=False)

    @functools.partial(pl.kernel, out_type=out_type, mesh=mesh,
                       scratch_types=scratch_types, compiler_params=cp)
    def gather_kernel(tokh, w0h, w1h, w2h, e0h, e1h, e2h,
                      tokv, g0v, s0v, g1v, s1v, g2v, s2v,
                      buf0, buf1, buf2):
        wid = jax.lax.axis_index("s") * NC + jax.lax.axis_index("c")
        base = wid * TOK_W

        pltpu.sync_copy(tokh.at[pl.ds(base, TOK_W)], tokv)

        lanes = jax.lax.iota(jnp.int32, 16)
        tables = (
            (g0v, s0v, 0, CUT0, lambda t: t, w0h, e0h, buf0),
            (g1v, s1v, CUT0, CUT1, lambda t: t - CUT0, w1h, e1h, buf1),
            (g2v, s2v, CUT1, CUT2, lambda t: (t - CUT1) >> 1, w2h, e2h,
             buf2),
        )

        cnts = [jnp.int32(0), jnp.int32(0), jnp.int32(0)]
        for v in range(TOK_W // 16):
            tok = tokv[pl.ds(v * 16, 16)]
            gpos = base + v * 16 + lanes
            for t, (gv, sv, lo, hi, to_row, _, _, _) in enumerate(tables):
                m = jnp.logical_and(tok >= lo, tok < hi)
                mi = m.astype(jnp.int32)
                pos = cnts[t] + jnp.cumsum(mi) - 1
                plsc.store_scatter(gv, [pos >> CH_SHIFT, pos & CH_MASK],
                                   to_row(tok), mask=m)
                plsc.store_scatter(sv, [pos >> CH_SHIFT, pos & CH_MASK],
                                   gpos, mask=m)
                cnts[t] = cnts[t] + jnp.sum(mi)

        zeros16 = lanes * 0
        for t, (gv, sv, *_rest) in enumerate(tables):
            cnt = cnts[t]
            padded = ((cnt + CH - 1) >> CH_SHIFT) << CH_SHIFT
            gfirst = plsc.load_gather(gv, [zeros16, zeros16])
            sfirst = plsc.load_gather(sv, [zeros16, zeros16])
            for k in range(CH // 16):
                p = cnt + k * 16 + lanes
                pm = p < padded
                plsc.store_scatter(gv, [p >> CH_SHIFT, p & CH_MASK],
                                   gfirst, mask=pm)
                plsc.store_scatter(sv, [p >> CH_SHIFT, p & CH_MASK],
                                   sfirst, mask=pm)

        for t, (gv, sv, _, _, _, wh, eh, buf) in enumerate(tables):
            used = (cnts[t] + CH - 1) >> CH_SHIFT
            for c in range(NCHUNK):
                @pl.when(c < used)
                def _():
                    pltpu.sync_copy(wh.at[gv.at[c]], buf)
                    pltpu.sync_copy(buf, eh.at[sv.at[c]])

    return gather_kernel(toks, w0, w1, w2.reshape(-1, 2 * D2))


def _tc_project_body(inp_ref, e0_ref, e1_ref, e2_ref, p0_ref, p1_ref, p2_ref,
                     out_ref):
    tok = inp_ref[...]
    m0 = tok < CUT0
    m1 = jnp.logical_and(tok >= CUT0, tok < CUT1)
    m2 = tok >= CUT1
    odd = jnp.equal(jnp.bitwise_and(tok, 1), 1)
    e2pair = e2_ref[...]
    e2 = jnp.where(odd, e2pair[:, D2:], e2pair[:, :D2])
    a0 = jnp.where(m0, e0_ref[...], 0.0).astype(jnp.bfloat16)
    a1 = jnp.where(m1, e1_ref[...], 0.0).astype(jnp.bfloat16)
    a2 = jnp.where(m2, e2, 0.0).astype(jnp.bfloat16)
    acc = jnp.dot(a0, p0_ref[...], preferred_element_type=jnp.float32)
    acc = acc + jnp.dot(a1, p1_ref[...], preferred_element_type=jnp.float32)
    acc = acc + jnp.dot(a2, p2_ref[...], preferred_element_type=jnp.float32)
    out_ref[...] = acc * (D_PROJ ** 0.5)


def _tc_project(inp2, e0, e1, e2, p0b, p1b, p2b):
    grid = (N_TOK_TOTAL // TOK_BLOCK,)
    return pl.pallas_call(
        _tc_project_body,
        grid=grid,
        in_specs=[
            pl.BlockSpec((TOK_BLOCK, 1), lambda i: (i, 0)),
            pl.BlockSpec((TOK_BLOCK, D0), lambda i: (i, 0)),
            pl.BlockSpec((TOK_BLOCK, D1), lambda i: (i, 0)),
            pl.BlockSpec((TOK_BLOCK, 2 * D2), lambda i: (i, 0)),
            pl.BlockSpec((D0, D_PROJ), lambda i: (0, 0)),
            pl.BlockSpec((D1, D_PROJ), lambda i: (0, 0)),
            pl.BlockSpec((D2, D_PROJ), lambda i: (0, 0)),
        ],
        out_specs=pl.BlockSpec((TOK_BLOCK, D_PROJ), lambda i: (i, 0)),
        out_shape=jax.ShapeDtypeStruct((N_TOK_TOTAL, D_PROJ), jnp.float32),
        compiler_params=pltpu.CompilerParams(
            dimension_semantics=("parallel",)),
    )(inp2, e0, e1, e2, p0b, p1b, p2b)


def kernel(inp, w0, w1, w2, p0, p1, p2):
    inp_flat = inp.reshape(-1)
    e0, e1, e2 = _sc_gather(inp_flat, w0, w1, w2)

    inp2 = inp_flat.reshape(-1, 1)
    out = _tc_project(inp2, e0, e1, e2,
                      p0.astype(jnp.bfloat16),
                      p1.astype(jnp.bfloat16),
                      p2.astype(jnp.bfloat16))
    return out.reshape(inp.shape + (D_PROJ,))


# speedup vs baseline: 8.2318x; 1.0396x over previous
import dataclasses
import functools

import jax
import jax.numpy as jnp
from jax.experimental import pallas as pl
from jax.experimental.pallas import tpu as pltpu
from jax.experimental.pallas import tpu_sc as plsc

N_TOKEN = 100000
D_PROJ = 1024
CUT0, CUT1, CUT2 = 20000, 60000, 100000
D0, D1, D2 = 1024, 256, 64
N_TOK_TOTAL = 8192

TOK_BLOCK = 1024

NC, NS = 2, 16
NW = NC * NS
TOK_W = N_TOK_TOTAL // NW

CH = 64
NCHUNK = TOK_W // CH
CH_SHIFT, CH_MASK = 6, CH - 1


def _sc_gather(toks, w0, w1, w2):
    mesh = plsc.VectorSubcoreMesh(core_axis_name="c", subcore_axis_name="s")

    out_type = (
        jax.ShapeDtypeStruct((N_TOK_TOTAL, D0), jnp.float32),
        jax.ShapeDtypeStruct((N_TOK_TOTAL, D1), jnp.float32),
        jax.ShapeDtypeStruct((N_TOK_TOTAL, 2 * D2), jnp.float32),
    )

    scratch_types = [
        pltpu.VMEM((TOK_W,), jnp.int32),
        pltpu.VMEM((NCHUNK, CH), jnp.int32),
        pltpu.VMEM((NCHUNK, CH), jnp.int32),
        pltpu.VMEM((NCHUNK, CH), jnp.int32),
        pltpu.VMEM((NCHUNK, CH), jnp.int32),
        pltpu.VMEM((NCHUNK, CH), jnp.int32),
        pltpu.VMEM((NCHUNK, CH), jnp.int32),
        pltpu.VMEM((CH, D0), jnp.float32),
        pltpu.VMEM((CH, D1), jnp.float32),
        pltpu.VMEM((CH, 2 * D2), jnp.float32),
    ]

    cp = pltpu.CompilerParams()
    if "needs_layout_passes" in pltpu.CompilerParams.__dataclass_fields__:
        cp = dataclasses.replace(cp, needs_layout_passes=False)

    @functools.partial(pl.kernel, out_type=out_type, mesh=mesh,
                       scratch_types=scratch_types, compiler_params=cp)
    def gather_kernel(tokh, w0h, w1h, w2h, e0h, e1h, e2h,
                      tokv, g0v, s0v, g1v, s1v, g2v, s2v,
                      buf0, buf1, buf2):
        wid = jax.lax.axis_index("s") * NC + jax.lax.axis_index("c")
        base = wid * TOK_W

        pltpu.sync_copy(tokh.at[pl.ds(base, TOK_W)], tokv)

        lanes = jax.lax.iota(jnp.int32, 16)
        tables = (
            (g0v, s0v, 0, CUT0, lambda t: t, w0h, e0h, buf0),
            (g1v, s1v, CUT0, CUT1, lambda t: t - CUT0, w1h, e1h, buf1),
            (g2v, s2v, CUT1, CUT2, lambda t: (t - CUT1) >> 1, w2h, e2h,
             buf2),
        )

        cnts = [jnp.int32(0), jnp.int32(0), jnp.int32(0)]
        for v in range(TOK_W // 16):
            tok = tokv[pl.ds(v * 16, 16)]
            gpos = base + v * 16 + lanes
            for t, (gv, sv, lo, hi, to_row, _, _, _) in enumerate(tables):
                m = jnp.logical_and(tok >= lo, tok < hi)
                mi = m.astype(jnp.int32)
                pos = cnts[t] + jnp.cumsum(mi) - 1
                plsc.store_scatter(gv, [pos >> CH_SHIFT, pos & CH_MASK],
                                   to_row(tok), mask=m)
                plsc.store_scatter(sv, [pos >> CH_SHIFT, pos & CH_MASK],
                                   gpos, mask=m)
                cnts[t] = cnts[t] + jnp.sum(mi)

        zeros16 = lanes * 0
        for t, (gv, sv, *_rest) in enumerate(tables):
            cnt = cnts[t]
            padded = ((cnt + CH - 1) >> CH_SHIFT) << CH_SHIFT
            gfirst = plsc.load_gather(gv, [zeros16, zeros16])
            sfirst = plsc.load_gather(sv, [zeros16, zeros16])
            for k in range(CH // 16):
                p = cnt + k * 16 + lanes
                pm = p < padded
                plsc.store_scatter(gv, [p >> CH_SHIFT, p & CH_MASK],
                                   gfirst, mask=pm)
                plsc.store_scatter(sv, [p >> CH_SHIFT, p & CH_MASK],
                                   sfirst, mask=pm)

        for t, (gv, sv, _, _, _, wh, eh, buf) in enumerate(tables):
            used = (cnts[t] + CH - 1) >> CH_SHIFT
            for c in range(NCHUNK):
                @pl.when(c < used)
                def _():
                    pltpu.sync_copy(wh.at[gv.at[c]], buf)
                    pltpu.sync_copy(buf, eh.at[sv.at[c]])

    return gather_kernel(toks, w0, w1, w2.reshape(-1, 2 * D2))


def _tc_project_body(inp_ref, e0_ref, e1_ref, e2_ref, p0_ref, p1_ref, p2_ref,
                     out_ref):
    tok = inp_ref[...]
    m0 = tok < CUT0
    m1 = jnp.logical_and(tok >= CUT0, tok < CUT1)
    m2 = tok >= CUT1
    odd = jnp.equal(jnp.bitwise_and(tok, 1), 1)
    e2pair = e2_ref[...]
    e2 = jnp.where(odd, e2pair[:, D2:], e2pair[:, :D2])
    a0 = jnp.where(m0, e0_ref[...], 0.0).astype(jnp.bfloat16)
    a1 = jnp.where(m1, e1_ref[...], 0.0).astype(jnp.bfloat16)
    a2 = jnp.where(m2, e2, 0.0).astype(jnp.bfloat16)
    acc = jnp.dot(a0, p0_ref[...], preferred_element_type=jnp.float32)
    acc = acc + jnp.dot(a1, p1_ref[...], preferred_element_type=jnp.float32)
    acc = acc + jnp.dot(a2, p2_ref[...], preferred_element_type=jnp.float32)
    out_ref[...] = acc


def _tc_project(inp2, e0, e1, e2, p0b, p1b, p2b):
    grid = (N_TOK_TOTAL // TOK_BLOCK,)
    return pl.pallas_call(
        _tc_project_body,
        grid=grid,
        in_specs=[
            pl.BlockSpec((TOK_BLOCK, 1), lambda i: (i, 0)),
            pl.BlockSpec((TOK_BLOCK, D0), lambda i: (i, 0)),
            pl.BlockSpec((TOK_BLOCK, D1), lambda i: (i, 0)),
            pl.BlockSpec((TOK_BLOCK, 2 * D2), lambda i: (i, 0)),
            pl.BlockSpec((D0, D_PROJ), lambda i: (0, 0)),
            pl.BlockSpec((D1, D_PROJ), lambda i: (0, 0)),
            pl.BlockSpec((D2, D_PROJ), lambda i: (0, 0)),
        ],
        out_specs=pl.BlockSpec((TOK_BLOCK, D_PROJ), lambda i: (i, 0)),
        out_shape=jax.ShapeDtypeStruct((N_TOK_TOTAL, D_PROJ), jnp.float32),
        compiler_params=pltpu.CompilerParams(
            dimension_semantics=("parallel",)),
    )(inp2, e0, e1, e2, p0b, p1b, p2b)


def kernel(inp, w0, w1, w2, p0, p1, p2):
    inp_flat = inp.reshape(-1)
    e0, e1, e2 = _sc_gather(inp_flat, w0, w1, w2)

    inp2 = inp_flat.reshape(-1, 1)
    scale = D_PROJ ** 0.5
    out = _tc_project(inp2, e0, e1, e2,
                      (p0 * scale).astype(jnp.bfloat16),
                      (p1 * scale).astype(jnp.bfloat16),
                      (p2 * scale).astype(jnp.bfloat16))
    return out.reshape(inp.shape + (D_PROJ,))


# speedup vs baseline: 8.2893x; 1.0070x over previous
import dataclasses
import functools

import jax
import jax.numpy as jnp
from jax.experimental import pallas as pl
from jax.experimental.pallas import tpu as pltpu
from jax.experimental.pallas import tpu_sc as plsc

N_TOKEN = 100000
D_PROJ = 1024
CUT0, CUT1, CUT2 = 20000, 60000, 100000
D0, D1, D2 = 1024, 256, 64
N_TOK_TOTAL = 8192

TOK_BLOCK = 1024

NC, NS = 2, 16
NW = NC * NS
TOK_W = N_TOK_TOTAL // NW

CH = 64
NCHUNK = TOK_W // CH
CH_SHIFT, CH_MASK = 6, CH - 1


def _sc_gather(toks, w0, w1, w2):
    mesh = plsc.VectorSubcoreMesh(core_axis_name="c", subcore_axis_name="s")

    out_type = (
        jax.ShapeDtypeStruct((N_TOK_TOTAL, D0), jnp.float32),
        jax.ShapeDtypeStruct((N_TOK_TOTAL, D1), jnp.float32),
        jax.ShapeDtypeStruct((N_TOK_TOTAL, 2 * D2), jnp.float32),
    )

    scratch_types = [
        pltpu.VMEM((TOK_W,), jnp.int32),
        pltpu.VMEM((NCHUNK, CH), jnp.int32),
        pltpu.VMEM((NCHUNK, CH), jnp.int32),
        pltpu.VMEM((NCHUNK, CH), jnp.int32),
        pltpu.VMEM((NCHUNK, CH), jnp.int32),
        pltpu.VMEM((NCHUNK, CH), jnp.int32),
        pltpu.VMEM((NCHUNK, CH), jnp.int32),
        pltpu.VMEM((CH, D0), jnp.float32),
        pltpu.VMEM((CH, D1), jnp.float32),
        pltpu.VMEM((CH, 2 * D2), jnp.float32),
    ]

    cp = pltpu.CompilerParams()
    if "needs_layout_passes" in pltpu.CompilerParams.__dataclass_fields__:
        cp = dataclasses.replace(cp, needs_layout_passes=False)

    @functools.partial(pl.kernel, out_type=out_type, mesh=mesh,
                       scratch_types=scratch_types, compiler_params=cp)
    def gather_kernel(tokh, w0h, w1h, w2h, e0h, e1h, e2h,
                      tokv, g0v, s0v, g1v, s1v, g2v, s2v,
                      buf0, buf1, buf2):
        wid = jax.lax.axis_index("s") * NC + jax.lax.axis_index("c")
        base = wid * TOK_W

        per_row = 2048 // TOK_W
        pltpu.sync_copy(
            tokh.at[wid // per_row, pl.ds((wid % per_row) * TOK_W, TOK_W)],
            tokv)

        lanes = jax.lax.iota(jnp.int32, 16)
        tables = (
            (g0v, s0v, 0, CUT0, lambda t: t, w0h, e0h, buf0),
            (g1v, s1v, CUT0, CUT1, lambda t: t - CUT0, w1h, e1h, buf1),
            (g2v, s2v, CUT1, CUT2, lambda t: (t - CUT1) >> 1, w2h, e2h,
             buf2),
        )

        cnts = [jnp.int32(0), jnp.int32(0), jnp.int32(0)]
        for v in range(TOK_W // 16):
            tok = tokv[pl.ds(v * 16, 16)]
            gpos = base + v * 16 + lanes
            for t, (gv, sv, lo, hi, to_row, _, _, _) in enumerate(tables):
                m = jnp.logical_and(tok >= lo, tok < hi)
                mi = m.astype(jnp.int32)
                pos = cnts[t] + jnp.cumsum(mi) - 1
                plsc.store_scatter(gv, [pos >> CH_SHIFT, pos & CH_MASK],
                                   to_row(tok), mask=m)
                plsc.store_scatter(sv, [pos >> CH_SHIFT, pos & CH_MASK],
                                   gpos, mask=m)
                cnts[t] = cnts[t] + jnp.sum(mi)

        zeros16 = lanes * 0
        for t, (gv, sv, *_rest) in enumerate(tables):
            cnt = cnts[t]
            padded = ((cnt + CH - 1) >> CH_SHIFT) << CH_SHIFT
            gfirst = plsc.load_gather(gv, [zeros16, zeros16])
            sfirst = plsc.load_gather(sv, [zeros16, zeros16])
            for k in range(CH // 16):
                p = cnt + k * 16 + lanes
                pm = p < padded
                plsc.store_scatter(gv, [p >> CH_SHIFT, p & CH_MASK],
                                   gfirst, mask=pm)
                plsc.store_scatter(sv, [p >> CH_SHIFT, p & CH_MASK],
                                   sfirst, mask=pm)

        for t, (gv, sv, _, _, _, wh, eh, buf) in enumerate(tables):
            used = (cnts[t] + CH - 1) >> CH_SHIFT
            for c in range(NCHUNK):
                @pl.when(c < used)
                def _():
                    pltpu.sync_copy(wh.at[gv.at[c]], buf)
                    pltpu.sync_copy(buf, eh.at[sv.at[c]])

    return gather_kernel(toks, w0, w1, w2.reshape(-1, 2 * D2))


def _tc_project_body(inp_ref, e0_ref, e1_ref, e2_ref, p0_ref, p1_ref, p2_ref,
                     out_ref):
    i = pl.program_id(0)
    per_row = 2048 // TOK_BLOCK
    tokr = inp_ref[pl.ds(i // per_row, 1), pl.ds((i % per_row) * TOK_BLOCK,
                                                 TOK_BLOCK)]
    tok = jnp.transpose(tokr)
    m0 = tok < CUT0
    m1 = tok < CUT1
    a0 = e0_ref[...].astype(jnp.bfloat16)
    a1 = e1_ref[...].astype(jnp.bfloat16)
    odd = jnp.equal(jnp.bitwise_and(tok, 1), 1)
    e2pair = e2_ref[...]
    a2 = jnp.where(odd, e2pair[:, D2:], e2pair[:, :D2]).astype(jnp.bfloat16)
    d0 = jnp.dot(a0, p0_ref[...], preferred_element_type=jnp.float32)
    d1 = jnp.dot(a1, p1_ref[...], preferred_element_type=jnp.float32)
    d2 = jnp.dot(a2, p2_ref[...], preferred_element_type=jnp.float32)
    out_ref[...] = jnp.where(m0, d0, jnp.where(m1, d1, d2))


def _tc_project(inp, e0, e1, e2, p0b, p1b, p2b):
    grid = (N_TOK_TOTAL // TOK_BLOCK,)
    per_row = 2048 // TOK_BLOCK
    return pl.pallas_call(
        _tc_project_body,
        grid=grid,
        in_specs=[
            pl.BlockSpec((4, 2048), lambda i: (0, 0)),
            pl.BlockSpec((TOK_BLOCK, D0), lambda i: (i, 0)),
            pl.BlockSpec((TOK_BLOCK, D1), lambda i: (i, 0)),
            pl.BlockSpec((TOK_BLOCK, 2 * D2), lambda i: (i, 0)),
            pl.BlockSpec((D0, D_PROJ), lambda i: (0, 0)),
            pl.BlockSpec((D1, D_PROJ), lambda i: (0, 0)),
            pl.BlockSpec((D2, D_PROJ), lambda i: (0, 0)),
        ],
        out_specs=pl.BlockSpec((TOK_BLOCK, D_PROJ), lambda i: (i, 0)),
        out_shape=jax.ShapeDtypeStruct((N_TOK_TOTAL, D_PROJ), jnp.float32),
        compiler_params=pltpu.CompilerParams(
            dimension_semantics=("parallel",)),
    )(inp, e0, e1, e2, p0b, p1b, p2b)


def kernel(inp, w0, w1, w2, p0, p1, p2):
    e0, e1, e2 = _sc_gather(inp, w0, w1, w2)

    scale = D_PROJ ** 0.5
    out = _tc_project(inp, e0, e1, e2,
                      (p0 * scale).astype(jnp.bfloat16),
                      (p1 * scale).astype(jnp.bfloat16),
                      (p2 * scale).astype(jnp.bfloat16))
    return out.reshape(inp.shape + (D_PROJ,))


# speedup vs baseline: 8.4969x; 1.0250x over previous
import dataclasses
import functools

import jax
import jax.numpy as jnp
from jax.experimental import pallas as pl
from jax.experimental.pallas import tpu as pltpu
from jax.experimental.pallas import tpu_sc as plsc

N_TOKEN = 100000
D_PROJ = 1024
CUT0, CUT1, CUT2 = 20000, 60000, 100000
D0, D1, D2 = 1024, 256, 64
N_TOK_TOTAL = 8192

TOK_BLOCK = 1024

NC, NS = 2, 16
NW = NC * NS
TOK_W = N_TOK_TOTAL // NW

CH = 64
NCHUNK = TOK_W // CH
CH_SHIFT, CH_MASK = 6, CH - 1


def _sc_gather(toks, w0, w1, w2):
    mesh = plsc.VectorSubcoreMesh(core_axis_name="c", subcore_axis_name="s")

    out_type = (
        jax.ShapeDtypeStruct((N_TOK_TOTAL, D0), jnp.float32),
        jax.ShapeDtypeStruct((N_TOK_TOTAL, D1), jnp.float32),
        jax.ShapeDtypeStruct((N_TOK_TOTAL, 2 * D2), jnp.float32),
    )

    scratch_types = [
        pltpu.VMEM((TOK_W,), jnp.int32),
        pltpu.VMEM((NCHUNK, CH), jnp.int32),
        pltpu.VMEM((NCHUNK, CH), jnp.int32),
        pltpu.VMEM((NCHUNK, CH), jnp.int32),
        pltpu.VMEM((NCHUNK, CH), jnp.int32),
        pltpu.VMEM((NCHUNK, CH), jnp.int32),
        pltpu.VMEM((NCHUNK, CH), jnp.int32),
        pltpu.VMEM((CH, D0), jnp.float32),
        pltpu.VMEM((CH, D1), jnp.float32),
        pltpu.VMEM((CH, 2 * D2), jnp.float32),
    ]

    cp = pltpu.CompilerParams()
    if "needs_layout_passes" in pltpu.CompilerParams.__dataclass_fields__:
        cp = dataclasses.replace(cp, needs_layout_passes=False)

    @functools.partial(pl.kernel, out_type=out_type, mesh=mesh,
                       scratch_types=scratch_types, compiler_params=cp)
    def gather_kernel(tokh, w0h, w1h, w2h, e0h, e1h, e2h,
                      tokv, g0v, s0v, g1v, s1v, g2v, s2v,
                      buf0, buf1, buf2):
        wid = jax.lax.axis_index("s") * NC + jax.lax.axis_index("c")
        base = wid * TOK_W

        per_row = 2048 // TOK_W
        pltpu.sync_copy(
            tokh.at[wid // per_row, pl.ds((wid % per_row) * TOK_W, TOK_W)],
            tokv)

        lanes = jax.lax.iota(jnp.int32, 16)
        tables = (
            (g0v, s0v, 0, CUT0, lambda t: t, w0h, e0h, buf0),
            (g1v, s1v, CUT0, CUT1, lambda t: t - CUT0, w1h, e1h, buf1),
            (g2v, s2v, CUT1, CUT2, lambda t: t - CUT1, w2h, e2h, buf2),
        )

        cnts = [jnp.int32(0), jnp.int32(0), jnp.int32(0)]
        for v in range(TOK_W // 16):
            tok = tokv[pl.ds(v * 16, 16)]
            gpos = base + v * 16 + lanes
            for t, (gv, sv, lo, hi, to_row, _, _, _) in enumerate(tables):
                m = jnp.logical_and(tok >= lo, tok < hi)
                mi = m.astype(jnp.int32)
                pos = cnts[t] + jnp.cumsum(mi) - 1
                plsc.store_scatter(gv, [pos >> CH_SHIFT, pos & CH_MASK],
                                   to_row(tok), mask=m)
                plsc.store_scatter(sv, [pos >> CH_SHIFT, pos & CH_MASK],
                                   gpos, mask=m)
                cnts[t] = cnts[t] + jnp.sum(mi)

        zeros16 = lanes * 0
        for t, (gv, sv, *_rest) in enumerate(tables):
            cnt = cnts[t]
            padded = ((cnt + CH - 1) >> CH_SHIFT) << CH_SHIFT
            gfirst = plsc.load_gather(gv, [zeros16, zeros16])
            sfirst = plsc.load_gather(sv, [zeros16, zeros16])
            for k in range(CH // 16):
                p = cnt + k * 16 + lanes
                pm = p < padded
                plsc.store_scatter(gv, [p >> CH_SHIFT, p & CH_MASK],
                                   gfirst, mask=pm)
                plsc.store_scatter(sv, [p >> CH_SHIFT, p & CH_MASK],
                                   sfirst, mask=pm)

        for t, (gv, sv, _, _, _, wh, eh, buf) in enumerate(tables):
            used = (cnts[t] + CH - 1) >> CH_SHIFT
            for c in range(NCHUNK):
                @pl.when(c < used)
                def _():
                    pltpu.sync_copy(wh.at[gv.at[c]], buf)
                    pltpu.sync_copy(buf, eh.at[sv.at[c]])

    return gather_kernel(toks, w0, w1, w2)


def _tc_project_body(inp_ref, e0_ref, e1_ref, e2_ref, p0_ref, p1_ref, p2_ref,
                     out_ref):
    i = pl.program_id(0)
    per_row = 2048 // TOK_BLOCK
    tokr = inp_ref[pl.ds(i // per_row, 1), pl.ds((i % per_row) * TOK_BLOCK,
                                                 TOK_BLOCK)]
    tok = jnp.transpose(tokr)
    m0 = tok < CUT0
    m1 = tok < CUT1
    a0 = e0_ref[...].astype(jnp.bfloat16)
    a1 = e1_ref[...].astype(jnp.bfloat16)
    a2 = e2_ref[:, :D2].astype(jnp.bfloat16)
    d0 = jnp.dot(a0, p0_ref[...], preferred_element_type=jnp.float32)
    d1 = jnp.dot(a1, p1_ref[...], preferred_element_type=jnp.float32)
    d2 = jnp.dot(a2, p2_ref[...], preferred_element_type=jnp.float32)
    out_ref[...] = jnp.where(m0, d0, jnp.where(m1, d1, d2))


def _tc_project(inp, e0, e1, e2, p0b, p1b, p2b):
    grid = (N_TOK_TOTAL // TOK_BLOCK,)
    per_row = 2048 // TOK_BLOCK
    return pl.pallas_call(
        _tc_project_body,
        grid=grid,
        in_specs=[
            pl.BlockSpec((4, 2048), lambda i: (0, 0)),
            pl.BlockSpec((TOK_BLOCK, D0), lambda i: (i, 0)),
            pl.BlockSpec((TOK_BLOCK, D1), lambda i: (i, 0)),
            pl.BlockSpec((TOK_BLOCK, 2 * D2), lambda i: (i, 0)),
            pl.BlockSpec((D0, D_PROJ), lambda i: (0, 0)),
            pl.BlockSpec((D1, D_PROJ), lambda i: (0, 0)),
            pl.BlockSpec((D2, D_PROJ), lambda i: (0, 0)),
        ],
        out_specs=pl.BlockSpec((TOK_BLOCK, D_PROJ), lambda i: (i, 0)),
        out_shape=jax.ShapeDtypeStruct((N_TOK_TOTAL, D_PROJ), jnp.float32),
        compiler_params=pltpu.CompilerParams(
            dimension_semantics=("parallel",)),
    )(inp, e0, e1, e2, p0b, p1b, p2b)


def kernel(inp, w0, w1, w2, p0, p1, p2):
    w2p = jnp.pad(w2, ((0, 0), (0, 2 * D2 - D2)))
    inp8 = jnp.pad(inp, ((0, 4), (0, 0)))
    e0, e1, e2 = _sc_gather(inp8, w0, w1, w2p)

    scale = D_PROJ ** 0.5
    out = _tc_project(inp, e0, e1, e2,
                      (p0 * scale).astype(jnp.bfloat16),
                      (p1 * scale).astype(jnp.bfloat16),
                      (p2 * scale).astype(jnp.bfloat16))
    return out.reshape(inp.shape + (D_PROJ,))
